# baseline (speedup 1.0000x reference)
import functools

import jax
import jax.numpy as jnp
from jax import lax
from jax.experimental import pallas as pl
from jax.experimental.pallas import tpu as pltpu
from jax.experimental.pallas import tpu_sc as plsc

NC = 2
NS = 16
NW = NC * NS
BATCH = 64
NSET = 2
NBUF = 2
NSLOT = NSET * NBUF


def _sc_partials(n_nodes, d_feat, n_edges):
    assert n_edges % BATCH == 0
    nbatch = n_edges // BATCH
    nb_lo = nbatch // NW
    n_extra = nbatch - nb_lo * NW
    assert nb_lo % NSLOT == 0
    rpt = (n_nodes // NS) // 8 * 8
    rtail = n_nodes - rpt * NS

    mesh = plsc.VectorSubcoreMesh(core_axis_name="c", subcore_axis_name="s")

    scratch = (
        [pltpu.VMEM_SHARED((n_nodes, d_feat), jnp.float32)]
        + [pltpu.VMEM((2, BATCH), jnp.int32) for _ in range(NSLOT)]
        + [pltpu.VMEM((BATCH, d_feat), jnp.float32) for _ in range(NSLOT)]
        + [pltpu.SemaphoreType.DMA for _ in range(3 * NSLOT)]
    )

    @functools.partial(
        pl.kernel,
        out_type=jax.ShapeDtypeStruct((NC, n_nodes, d_feat), jnp.float32),
        mesh=mesh,
        scratch_types=scratch,
    )
    def run(x_hbm, pk_hbm, zero_hbm, part_hbm, acc, *bufs):
        idx = bufs[:NSLOT]
        rows = bufs[NSLOT:2 * NSLOT]
        semi = bufs[2 * NSLOT:3 * NSLOT]
        semg = bufs[3 * NSLOT:4 * NSLOT]
        sems = bufs[4 * NSLOT:5 * NSLOT]
        c = lax.axis_index("c")
        s = lax.axis_index("s")
        w = c * NS + s
        start = w * nb_lo + jnp.minimum(w, n_extra)

        pltpu.sync_copy(zero_hbm.at[pl.ds(s * rpt, rpt)],
                        acc.at[pl.ds(s * rpt, rpt)])
        if rtail:
            @pl.when(s == 0)
            def _():
                pltpu.sync_copy(zero_hbm.at[pl.ds(rpt * NS, rtail)],
                                acc.at[pl.ds(rpt * NS, rtail)])
        plsc.subcore_barrier()

        def pair(k, _):
            base = start + k * NSLOT
            for st in range(NSET):
                sl = [st * NBUF + p for p in range(NBUF)]
                @pl.when(k > 0)
                def _():
                    for q in sl:
                        pltpu.make_async_copy(x_hbm.at[pl.ds(0, BATCH)],
                                              rows[q], sems[q]).wait()
                gi = [pltpu.async_copy(pk_hbm.at[base + q], idx[q], semi[q])
                      for q in sl]
                gg = []
                for i, q in enumerate(sl):
                    gi[i].wait()
                    gg.append(pltpu.async_copy(x_hbm.at[idx[q].at[0]],
                                               rows[q], semg[q]))
                for i, q in enumerate(sl):
                    gg[i].wait()
                    pltpu.async_copy(rows[q], acc.at[idx[q].at[1]],
                                     sems[q], add=True)
            return _

        lax.fori_loop(0, nb_lo // NSLOT, pair, None)
        for q in range(NSLOT):
            pltpu.make_async_copy(x_hbm.at[pl.ds(0, BATCH)],
                                  rows[q], sems[q]).wait()

        if n_extra:
            @pl.when(w < n_extra)
            def _():
                bb = start + nb_lo
                pltpu.async_copy(pk_hbm.at[bb], idx[0], semi[0]).wait()
                pltpu.async_copy(x_hbm.at[idx[0].at[0]], rows[0],
                                 semg[0]).wait()
                pltpu.async_copy(rows[0], acc.at[idx[0].at[1]], sems[0],
                                 add=True).wait()

        plsc.subcore_barrier()
        pltpu.sync_copy(acc.at[pl.ds(s * rpt, rpt)],
                        part_hbm.at[c, pl.ds(s * rpt, rpt)])
        if rtail:
            @pl.when(s == 0)
            def _():
                pltpu.sync_copy(acc.at[pl.ds(rpt * NS, rtail)],
                                part_hbm.at[c, pl.ds(rpt * NS, rtail)])

    return run


def _tc_add(a, b):
    n_nodes, d_feat = a.shape
    blk = 1000
    grid = n_nodes // blk

    def body(a_ref, b_ref, o_ref):
        o_ref[...] = a_ref[...] + b_ref[...]

    return pl.pallas_call(
        body,
        grid=(grid,),
        in_specs=[pl.BlockSpec((blk, d_feat), lambda i: (i, 0))] * 2,
        out_specs=pl.BlockSpec((blk, d_feat), lambda i: (i, 0)),
        out_shape=jax.ShapeDtypeStruct((n_nodes, d_feat), jnp.float32),
    )(a, b)


def kernel(boundary_x, boundary_index, out_size):
    n_nodes, d_feat = boundary_x.shape
    n_edges = boundary_index.shape[1]
    nbatch = n_edges // BATCH
    packed = boundary_index.astype(jnp.int32).reshape(2, nbatch, BATCH)
    packed = packed.transpose(1, 0, 2)
    zeros = jnp.zeros((n_nodes, d_feat), jnp.float32)
    part = _sc_partials(n_nodes, d_feat, n_edges)(boundary_x, packed, zeros)
    return _tc_add(part[0], part[1])


# speedup vs baseline: 1.3123x; 1.3123x over previous
import functools

import jax
import jax.numpy as jnp
from jax import lax
from jax.experimental import pallas as pl
from jax.experimental.pallas import tpu as pltpu
from jax.experimental.pallas import tpu_sc as plsc

NC = 2
NS = 16
NW = NC * NS
BATCH = 128
NROW = 3
NIDX = 6


def _sc_partials(n_nodes, d_feat, n_edges):
    assert n_edges % BATCH == 0
    nbatch = n_edges // BATCH
    nb_lo = nbatch // NW
    n_extra = nbatch - nb_lo * NW
    assert nb_lo % NIDX == 0
    ngroups = nb_lo // NIDX
    rpt = (n_nodes // NS) // 8 * 8
    rtail = n_nodes - rpt * NS

    mesh = plsc.VectorSubcoreMesh(core_axis_name="c", subcore_axis_name="s")

    scratch = (
        [pltpu.VMEM_SHARED((n_nodes, d_feat), jnp.float32)]
        + [pltpu.VMEM((2, BATCH), jnp.int32) for _ in range(NIDX)]
        + [pltpu.VMEM((BATCH, d_feat), jnp.float32) for _ in range(NROW)]
        + [pltpu.SemaphoreType.DMA for _ in range(NIDX + 2 * NROW)]
    )

    @functools.partial(
        pl.kernel,
        out_type=jax.ShapeDtypeStruct((NC, n_nodes, d_feat), jnp.float32),
        mesh=mesh,
        scratch_types=scratch,
    )
    def run(x_hbm, pk_hbm, zero_hbm, part_hbm, acc, *bufs):
        idx = bufs[:NIDX]
        rows = bufs[NIDX:NIDX + NROW]
        semi = bufs[NIDX + NROW:2 * NIDX + NROW]
        semg = bufs[2 * NIDX + NROW:2 * NIDX + 2 * NROW]
        sems = bufs[2 * NIDX + 2 * NROW:2 * NIDX + 3 * NROW]
        c = lax.axis_index("c")
        s = lax.axis_index("s")
        w = c * NS + s
        start = w * nb_lo + jnp.minimum(w, n_extra)

        pltpu.sync_copy(zero_hbm.at[pl.ds(s * rpt, rpt)],
                        acc.at[pl.ds(s * rpt, rpt)])
        if rtail:
            @pl.when(s == 0)
            def _():
                pltpu.sync_copy(zero_hbm.at[pl.ds(rpt * NS, rtail)],
                                acc.at[pl.ds(rpt * NS, rtail)])
        plsc.subcore_barrier()

        for p in range(2):
            pltpu.async_copy(pk_hbm.at[start + p], idx[p], semi[p])

        def group(g, _):
            base = start + g * NIDX
            for p in range(NIDX):
                pr = p % NROW
                if p < NROW:
                    @pl.when(g > 0)
                    def _():
                        pltpu.make_async_copy(x_hbm.at[pl.ds(0, BATCH)],
                                              rows[pr], sems[pr]).wait()
                else:
                    pltpu.make_async_copy(x_hbm.at[pl.ds(0, BATCH)],
                                          rows[pr], sems[pr]).wait()
                pf = (p + 2) % NIDX
                if p < NIDX - 2:
                    pltpu.async_copy(pk_hbm.at[base + p + 2], idx[pf],
                                     semi[pf])
                else:
                    @pl.when(g < ngroups - 1)
                    def _():
                        pltpu.async_copy(pk_hbm.at[base + p + 2], idx[pf],
                                         semi[pf])
                pltpu.make_async_copy(pk_hbm.at[base + p], idx[p],
                                      semi[p]).wait()
                pltpu.async_copy(x_hbm.at[idx[p].at[0]], rows[pr], semg[pr])
                pltpu.make_async_copy(x_hbm.at[pl.ds(0, BATCH)], rows[pr],
                                      semg[pr]).wait()
                pltpu.async_copy(rows[pr], acc.at[idx[p].at[1]], sems[pr],
                                 add=True)
            return _

        lax.fori_loop(0, ngroups, group, None)
        for pr in range(NROW):
            pltpu.make_async_copy(x_hbm.at[pl.ds(0, BATCH)],
                                  rows[pr], sems[pr]).wait()

        if n_extra:
            @pl.when(w < n_extra)
            def _():
                bb = start + nb_lo
                pltpu.async_copy(pk_hbm.at[bb], idx[0], semi[0]).wait()
                pltpu.async_copy(x_hbm.at[idx[0].at[0]], rows[0],
                                 semg[0]).wait()
                pltpu.async_copy(rows[0], acc.at[idx[0].at[1]], sems[0],
                                 add=True).wait()

        plsc.subcore_barrier()
        pltpu.sync_copy(acc.at[pl.ds(s * rpt, rpt)],
                        part_hbm.at[c, pl.ds(s * rpt, rpt)])
        if rtail:
            @pl.when(s == 0)
            def _():
                pltpu.sync_copy(acc.at[pl.ds(rpt * NS, rtail)],
                                part_hbm.at[c, pl.ds(rpt * NS, rtail)])

    return run


def _tc_add(a, b):
    n_nodes, d_feat = a.shape
    blk = 1000
    grid = n_nodes // blk

    def body(a_ref, b_ref, o_ref):
        o_ref[...] = a_ref[...] + b_ref[...]

    return pl.pallas_call(
        body,
        grid=(grid,),
        in_specs=[pl.BlockSpec((blk, d_feat), lambda i: (i, 0))] * 2,
        out_specs=pl.BlockSpec((blk, d_feat), lambda i: (i, 0)),
        out_shape=jax.ShapeDtypeStruct((n_nodes, d_feat), jnp.float32),
    )(a, b)


def kernel(boundary_x, boundary_index, out_size):
    n_nodes, d_feat = boundary_x.shape
    n_edges = boundary_index.shape[1]
    nbatch = n_edges // BATCH
    packed = boundary_index.astype(jnp.int32).reshape(2, nbatch, BATCH)
    packed = packed.transpose(1, 0, 2)
    zeros = jnp.zeros((n_nodes, d_feat), jnp.float32)
    part = _sc_partials(n_nodes, d_feat, n_edges)(boundary_x, packed, zeros)
    return _tc_add(part[0], part[1])
